# baseline (speedup 1.0000x reference)
import jax
import jax.numpy as jnp
from jax.experimental import pallas as pl
from jax.experimental.pallas import tpu as pltpu

_N_CHUNKS = 16


def kernel(x, queue):
    del queue
    B, D = x.shape
    ch = B // _N_CHUNKS

    def body(x_hbm, o_hbm, buf, in_sems, out_sems):
        reads = [
            pltpu.make_async_copy(
                x_hbm.at[pl.ds(i * ch, ch)], buf.at[pl.ds(i * ch, ch)],
                in_sems.at[i])
            for i in range(_N_CHUNKS)
        ]
        writes = [
            pltpu.make_async_copy(
                buf.at[pl.ds(i * ch, ch)], o_hbm.at[pl.ds(i * ch, ch)],
                out_sems.at[i])
            for i in range(_N_CHUNKS)
        ]
        for r in reads:
            r.start()
        for r, w in zip(reads, writes):
            r.wait()
            w.start()
        for w in writes:
            w.wait()

    return pl.pallas_call(
        body,
        in_specs=[pl.BlockSpec(memory_space=pl.ANY)],
        out_specs=pl.BlockSpec(memory_space=pl.ANY),
        out_shape=jax.ShapeDtypeStruct((B, D), x.dtype),
        scratch_shapes=[
            pltpu.VMEM((B, D), x.dtype),
            pltpu.SemaphoreType.DMA((_N_CHUNKS,)),
            pltpu.SemaphoreType.DMA((_N_CHUNKS,)),
        ],
    )(x)


# speedup vs baseline: 1.0491x; 1.0491x over previous
import jax
import jax.numpy as jnp
from jax.experimental import pallas as pl
from jax.experimental.pallas import tpu as pltpu

_N_CHUNKS = 4


def kernel(x, queue):
    del queue
    B, D = x.shape
    ch = B // _N_CHUNKS

    def body(x_hbm, o_hbm, buf, in_sems, out_sems):
        reads = [
            pltpu.make_async_copy(
                x_hbm.at[pl.ds(i * ch, ch)], buf.at[pl.ds(i * ch, ch)],
                in_sems.at[i])
            for i in range(_N_CHUNKS)
        ]
        writes = [
            pltpu.make_async_copy(
                buf.at[pl.ds(i * ch, ch)], o_hbm.at[pl.ds(i * ch, ch)],
                out_sems.at[i])
            for i in range(_N_CHUNKS)
        ]
        for r in reads:
            r.start()
        for r, w in zip(reads, writes):
            r.wait()
            w.start()
        for w in writes:
            w.wait()

    return pl.pallas_call(
        body,
        in_specs=[pl.BlockSpec(memory_space=pl.ANY)],
        out_specs=pl.BlockSpec(memory_space=pl.ANY),
        out_shape=jax.ShapeDtypeStruct((B, D), x.dtype),
        scratch_shapes=[
            pltpu.VMEM((B, D), x.dtype),
            pltpu.SemaphoreType.DMA((_N_CHUNKS,)),
            pltpu.SemaphoreType.DMA((_N_CHUNKS,)),
        ],
    )(x)
